# baseline (speedup 1.0000x reference)
import jax
import jax.numpy as jnp
from jax.experimental import pallas as pl
from jax.experimental.pallas import tpu as pltpu

_CH = 16
_EMB = 128
_HB = 8
_W = 384


def _shift_l(v):
    return jnp.concatenate([v[..., 1:], jnp.zeros_like(v[..., :1])], axis=-1)


def _shift_r(v):
    return jnp.concatenate([jnp.zeros_like(v[..., :1]), v[..., :-1]], axis=-1)


def _fused_kernel(xp, xc, xn, fm, w1, b1, w2, b2, w3, b3, out):
    xprev = xp[0, :, _HB - 1:_HB, :]
    xcur = xc[0]
    xnext = xn[0, :, 0:1, :]
    xe = jnp.concatenate([xprev, xcur, xnext], axis=1)

    up = xe[:, :-2, :]
    mid = xe[:, 1:-1, :]
    dn = xe[:, 2:, :]
    upl, upr = _shift_l(up), _shift_r(up)
    midl, midr = _shift_l(mid), _shift_r(mid)
    dnl, dnr = _shift_l(dn), _shift_r(dn)
    gx = (upl - upr) + 2.0 * (midl - midr) + (dnl - dnr)
    gy = (dnl + 2.0 * dn + dnr) - (upl + 2.0 * up + upr)

    a_up, a_mid, a_dn = up[3], mid[3], dn[3]
    pm = jnp.maximum(jnp.maximum(a_up, a_mid), a_dn)
    pooled = jnp.maximum(jnp.maximum(_shift_l(pm), pm), _shift_r(pm))
    act = jnp.where((pooled > 0.1) & (fm[0, 0] != 0), 1.0, 0.0)

    n = _HB * _W
    f = jnp.concatenate([xcur, gx, gy], axis=0).reshape(3 * _CH, n)
    prec = jax.lax.Precision.HIGHEST
    h1 = jnp.maximum(
        jnp.dot(w1[...], f, preferred_element_type=jnp.float32, precision=prec)
        + b1[...], 0.0)
    h2 = jnp.maximum(
        jnp.dot(w2[...], h1, preferred_element_type=jnp.float32, precision=prec)
        + b2[...], 0.0)
    u = (jnp.dot(w3[...], h2, preferred_element_type=jnp.float32, precision=prec)
         + b3[...])
    out[0] = (u * act.reshape(1, n)).reshape(_CH, _HB, _W)


def kernel(x, fire_mask, W1, b1, W2, b2, W3, b3):
    B, C, H, W = x.shape
    nh = H // _HB
    xpad = jnp.pad(x, ((0, 0), (0, 0), (_HB, _HB), (0, 0)))
    b1c = b1.reshape(_EMB, 1)
    b2c = b2.reshape(_EMB, 1)
    b3c = b3.reshape(_CH, 1)

    def spec_x(off):
        return pl.BlockSpec((1, C, _HB, W), lambda b, h: (b, 0, h + off, 0))

    return pl.pallas_call(
        _fused_kernel,
        grid=(B, nh),
        in_specs=[
            spec_x(0), spec_x(1), spec_x(2),
            pl.BlockSpec((1, 1, _HB, W), lambda b, h: (b, 0, h, 0)),
            pl.BlockSpec((_EMB, 3 * _CH), lambda b, h: (0, 0)),
            pl.BlockSpec((_EMB, 1), lambda b, h: (0, 0)),
            pl.BlockSpec((_EMB, _EMB), lambda b, h: (0, 0)),
            pl.BlockSpec((_EMB, 1), lambda b, h: (0, 0)),
            pl.BlockSpec((_CH, _EMB), lambda b, h: (0, 0)),
            pl.BlockSpec((_CH, 1), lambda b, h: (0, 0)),
        ],
        out_specs=pl.BlockSpec((1, C, _HB, W), lambda b, h: (b, 0, h, 0)),
        out_shape=jax.ShapeDtypeStruct((B, C, H, W), jnp.float32),
        compiler_params=pltpu.CompilerParams(
            dimension_semantics=("parallel", "parallel")),
    )(xpad, xpad, xpad, fire_mask, W1, b1c, W2, b2c, W3, b3c)


# speedup vs baseline: 2.8700x; 2.8700x over previous
import jax
import jax.numpy as jnp
from jax.experimental import pallas as pl
from jax.experimental.pallas import tpu as pltpu

_CH = 16
_EMB = 128
_HB = 8
_W = 384


def _shift_l(v):
    return jnp.concatenate([v[..., 1:], jnp.zeros_like(v[..., :1])], axis=-1)


def _shift_r(v):
    return jnp.concatenate([jnp.zeros_like(v[..., :1]), v[..., :-1]], axis=-1)


def _fused_kernel(xp, xc, xn, fm, w1, b1, w2, b2, w3, b3, out):
    xprev = xp[0, :, _HB - 1:_HB, :]
    xcur = xc[0]
    xnext = xn[0, :, 0:1, :]
    xe = jnp.concatenate([xprev, xcur, xnext], axis=1)

    up = xe[:, :-2, :]
    mid = xe[:, 1:-1, :]
    dn = xe[:, 2:, :]
    upl, upr = _shift_l(up), _shift_r(up)
    midl, midr = _shift_l(mid), _shift_r(mid)
    dnl, dnr = _shift_l(dn), _shift_r(dn)
    gx = (upl - upr) + 2.0 * (midl - midr) + (dnl - dnr)
    gy = (dnl + 2.0 * dn + dnr) - (upl + 2.0 * up + upr)

    a_up, a_mid, a_dn = up[3], mid[3], dn[3]
    pm = jnp.maximum(jnp.maximum(a_up, a_mid), a_dn)
    pooled = jnp.maximum(jnp.maximum(_shift_l(pm), pm), _shift_r(pm))
    act = jnp.where((pooled > 0.1) & (fm[0, 0] != 0), 1.0, 0.0)

    n = _HB * _W
    f = jnp.concatenate([xcur, gx, gy], axis=0).reshape(3 * _CH, n)
    prec = jax.lax.Precision.DEFAULT
    h1 = jnp.maximum(
        jnp.dot(w1[...], f, preferred_element_type=jnp.float32, precision=prec)
        + b1[...], 0.0)
    h2 = jnp.maximum(
        jnp.dot(w2[...], h1, preferred_element_type=jnp.float32, precision=prec)
        + b2[...], 0.0)
    u = (jnp.dot(w3[...], h2, preferred_element_type=jnp.float32, precision=prec)
         + b3[...])
    out[0] = (u * act.reshape(1, n)).reshape(_CH, _HB, _W)


def kernel(x, fire_mask, W1, b1, W2, b2, W3, b3):
    B, C, H, W = x.shape
    nh = H // _HB
    xpad = jnp.pad(x, ((0, 0), (0, 0), (_HB, _HB), (0, 0)))
    b1c = b1.reshape(_EMB, 1)
    b2c = b2.reshape(_EMB, 1)
    b3c = b3.reshape(_CH, 1)

    def spec_x(off):
        return pl.BlockSpec((1, C, _HB, W), lambda b, h: (b, 0, h + off, 0))

    return pl.pallas_call(
        _fused_kernel,
        grid=(B, nh),
        in_specs=[
            spec_x(0), spec_x(1), spec_x(2),
            pl.BlockSpec((1, 1, _HB, W), lambda b, h: (b, 0, h, 0)),
            pl.BlockSpec((_EMB, 3 * _CH), lambda b, h: (0, 0)),
            pl.BlockSpec((_EMB, 1), lambda b, h: (0, 0)),
            pl.BlockSpec((_EMB, _EMB), lambda b, h: (0, 0)),
            pl.BlockSpec((_EMB, 1), lambda b, h: (0, 0)),
            pl.BlockSpec((_CH, _EMB), lambda b, h: (0, 0)),
            pl.BlockSpec((_CH, 1), lambda b, h: (0, 0)),
        ],
        out_specs=pl.BlockSpec((1, C, _HB, W), lambda b, h: (b, 0, h, 0)),
        out_shape=jax.ShapeDtypeStruct((B, C, H, W), jnp.float32),
        compiler_params=pltpu.CompilerParams(
            dimension_semantics=("parallel", "parallel")),
    )(xpad, xpad, xpad, fire_mask, W1, b1c, W2, b2c, W3, b3c)
